# baseline (speedup 1.0000x reference)
import functools

import jax
import jax.numpy as jnp
from jax import lax
from jax.experimental import pallas as pl
from jax.experimental.pallas import tpu as pltpu
from jax.experimental.pallas import tpu_sc as plsc

TAU = 0.5
NS = 4
D = 768
N_TOKENS = 32768

BLK = 2048
L = 16
NWORK = 32


def _tc_body(s_ref, u_ref, w_ref, b_ref, logits_ref, g_ref):
    s = s_ref[...]
    w = w_ref[...]
    logits_t = jax.lax.dot_general(
        w, s, (((1,), (1,)), ((), ())),
        preferred_element_type=jnp.float32,
    ) + b_ref[...]
    logits_ref[...] = logits_t
    u = u_ref[...]
    gumbel = -jnp.log(-jnp.log(u))
    g_ref[...] = (logits_t + gumbel) / TAU


def _tc_logits(S_t, u_t, W, b2):
    n = S_t.shape[0]
    grid = (n // BLK,)
    return pl.pallas_call(
        _tc_body,
        grid=grid,
        in_specs=[
            pl.BlockSpec((BLK, D), lambda i: (i, 0)),
            pl.BlockSpec((NS, BLK), lambda i: (0, i)),
            pl.BlockSpec((NS, D), lambda i: (0, 0)),
            pl.BlockSpec((NS, 1), lambda i: (0, 0)),
        ],
        out_specs=[
            pl.BlockSpec((NS, BLK), lambda i: (0, i)),
            pl.BlockSpec((NS, BLK), lambda i: (0, i)),
        ],
        out_shape=[
            jax.ShapeDtypeStruct((NS, n), jnp.float32),
            jax.ShapeDtypeStruct((NS, n), jnp.float32),
        ],
    )(S_t, u_t, W, b2)


def _sc_router(g_t):
    n = g_t.shape[1]
    c = n // NWORK
    mesh = plsc.VectorSubcoreMesh(core_axis_name="c", subcore_axis_name="s")

    @functools.partial(
        pl.kernel,
        mesh=mesh,
        out_type=[
            jax.ShapeDtypeStruct((NS, n), jnp.float32),
            jax.ShapeDtypeStruct((NS, n), jnp.float32),
            jax.ShapeDtypeStruct((L,), jnp.int32),
        ],
        scratch_types=(
            [pltpu.VMEM((c,), jnp.float32) for _ in range(3 * NS)]
            + [pltpu.VMEM((L,), jnp.int32)]
        ),
    )
    def k(g_hbm, y_hbm, rp_hbm, sel_hbm,
          g0, g1, g2, g3, y0, y1, y2, y3, r0, r1, r2, r3, sel_v):
        wid = lax.axis_index("s") * 2 + lax.axis_index("c")
        base = wid * c
        gs = (g0, g1, g2, g3)
        ys = (y0, y1, y2, y3)
        rs = (r0, r1, r2, r3)
        for e in range(NS):
            pltpu.sync_copy(g_hbm.at[e, pl.ds(base, c)], gs[e])
        for i in range(c // L):
            sl = pl.ds(i * L, L)
            v = [gs[e][sl] for e in range(NS)]
            m = jnp.maximum(jnp.maximum(v[0], v[1]), jnp.maximum(v[2], v[3]))
            ex = [jnp.exp(v[e] - m) for e in range(NS)]
            s = (ex[0] + ex[1]) + (ex[2] + ex[3])
            y = [ex[e] / s for e in range(NS)]
            ym = jnp.maximum(jnp.maximum(y[0], y[1]), jnp.maximum(y[2], y[3]))
            idx = jnp.where(
                y[0] == ym, jnp.int32(0),
                jnp.where(y[1] == ym, jnp.int32(1),
                          jnp.where(y[2] == ym, jnp.int32(2), jnp.int32(3))))
            for e in range(NS):
                ys[e][sl] = y[e]
                oh = jnp.where(idx == e, jnp.float32(1.0), jnp.float32(0.0))
                rs[e][sl] = (oh - y[e]) + y[e]
            if i == 0:
                @pl.when(wid == 0)
                def _():
                    sel_v[...] = idx
        for e in range(NS):
            pltpu.sync_copy(ys[e], y_hbm.at[e, pl.ds(base, c)])
            pltpu.sync_copy(rs[e], rp_hbm.at[e, pl.ds(base, c)])

        @pl.when(wid == 0)
        def _():
            pltpu.sync_copy(sel_v, sel_hbm)

    return k(g_t)


def kernel(S_t, u_noise, W, b):
    u_t = u_noise.T
    b2 = b.reshape(NS, 1)
    logits_t, g_t = _tc_logits(S_t, u_t, W, b2)
    ysoft_t, rp_t, sel = _sc_router(g_t)
    return (rp_t.T, sel[0], logits_t.T, ysoft_t.T)


# speedup vs baseline: 1.5996x; 1.5996x over previous
import jax
import jax.numpy as jnp
from jax.experimental import pallas as pl
from jax.experimental.pallas import tpu as pltpu

TAU = 0.5
NUM_SPECIALISTS = 4
WORKSPACE_DIM = 768
N_TOKENS = 32768

BLK = 2048


def _router_body(s_ref, u_ref, w_ref, b_ref,
                 logits_ref, ysoft_ref, rp_ref, sel_ref):
    s = s_ref[...]
    w = w_ref[...]
    logits_t = jax.lax.dot_general(
        w, s, (((1,), (1,)), ((), ())),
        preferred_element_type=jnp.float32,
    ) + b_ref[...]
    logits_ref[...] = logits_t

    u = u_ref[...]
    gumbel = -jnp.log(-jnp.log(u))
    g = (logits_t + gumbel) / TAU
    m = jnp.max(g, axis=0, keepdims=True)
    e = jnp.exp(g - m)
    ysoft = e / jnp.sum(e, axis=0, keepdims=True)
    ysoft_ref[...] = ysoft

    idx = jnp.argmax(ysoft, axis=0)
    onehot = (jax.lax.broadcasted_iota(jnp.int32, (NUM_SPECIALISTS, BLK), 0)
              == idx[None, :]).astype(jnp.float32)
    rp_ref[...] = (onehot - ysoft) + ysoft

    @pl.when(pl.program_id(0) == 0)
    def _():
        s0 = rp_ref[0, 0]
        s1 = rp_ref[1, 0]
        s2 = rp_ref[2, 0]
        s3 = rp_ref[3, 0]
        bi = jnp.int32(0)
        bv = s0
        bi = jnp.where(s1 > bv, jnp.int32(1), bi)
        bv = jnp.maximum(bv, s1)
        bi = jnp.where(s2 > bv, jnp.int32(2), bi)
        bv = jnp.maximum(bv, s2)
        bi = jnp.where(s3 > bv, jnp.int32(3), bi)
        sel_ref[0, 0] = bi


def kernel(S_t, u_noise, W, b):
    n_tokens = S_t.shape[0]
    grid = (n_tokens // BLK,)
    u_t = u_noise.T
    b2 = b.reshape(NUM_SPECIALISTS, 1)

    logits_t, ysoft_t, rp_t, sel = pl.pallas_call(
        _router_body,
        grid=grid,
        in_specs=[
            pl.BlockSpec((BLK, WORKSPACE_DIM), lambda i: (i, 0)),
            pl.BlockSpec((NUM_SPECIALISTS, BLK), lambda i: (0, i)),
            pl.BlockSpec((NUM_SPECIALISTS, WORKSPACE_DIM), lambda i: (0, 0)),
            pl.BlockSpec((NUM_SPECIALISTS, 1), lambda i: (0, 0)),
        ],
        out_specs=[
            pl.BlockSpec((NUM_SPECIALISTS, BLK), lambda i: (0, i)),
            pl.BlockSpec((NUM_SPECIALISTS, BLK), lambda i: (0, i)),
            pl.BlockSpec((NUM_SPECIALISTS, BLK), lambda i: (0, i)),
            pl.BlockSpec((1, 1), lambda i: (0, 0),
                         memory_space=pltpu.SMEM),
        ],
        out_shape=[
            jax.ShapeDtypeStruct((NUM_SPECIALISTS, n_tokens), jnp.float32),
            jax.ShapeDtypeStruct((NUM_SPECIALISTS, n_tokens), jnp.float32),
            jax.ShapeDtypeStruct((NUM_SPECIALISTS, n_tokens), jnp.float32),
            jax.ShapeDtypeStruct((1, 1), jnp.int32),
        ],
    )(S_t, u_t, W, b2)

    return (rp_t.T, sel.reshape(()), logits_t.T, ysoft_t.T)
